# baseline (speedup 1.0000x reference)
import jax
import jax.numpy as jnp
from jax.experimental import pallas as pl
from jax.experimental.pallas import tpu as pltpu

B, S, D = 4, 4096, 1024
BLK = 256


def _body(in_ref, emb_ref, out_ref):
    out_ref[...] = in_ref[...] + emb_ref[...][None]


_add = pl.pallas_call(
    _body,
    out_shape=jax.ShapeDtypeStruct((B, S, D), jnp.float32),
    grid=(S // BLK,),
    in_specs=[
        pl.BlockSpec((B, BLK, D), lambda s: (0, s, 0)),
        pl.BlockSpec((BLK, D), lambda s: (s, 0)),
    ],
    out_specs=pl.BlockSpec((B, BLK, D), lambda s: (0, s, 0)),
    compiler_params=pltpu.CompilerParams(
        dimension_semantics=("arbitrary",),
    ),
)


def kernel(inputs, embedding):
    return _add(inputs, embedding)


# speedup vs baseline: 1.0279x; 1.0279x over previous
import jax
import jax.numpy as jnp
from jax.experimental import pallas as pl
from jax.experimental.pallas import tpu as pltpu

B, S, D = 4, 4096, 1024
BLK = 512


def _body(in_ref, emb_ref, out_ref):
    out_ref[...] = in_ref[...] + emb_ref[pl.ds(pl.program_id(0) * BLK, BLK)][None]


_add = pl.pallas_call(
    _body,
    out_shape=jax.ShapeDtypeStruct((B, S, D), jnp.float32),
    grid=(S // BLK,),
    in_specs=[
        pl.BlockSpec((B, BLK, D), lambda s: (0, s, 0)),
        pl.BlockSpec((S, D), lambda s: (0, 0)),
    ],
    out_specs=pl.BlockSpec((B, BLK, D), lambda s: (0, s, 0)),
    compiler_params=pltpu.CompilerParams(
        dimension_semantics=("arbitrary",),
    ),
)


def kernel(inputs, embedding):
    return _add(inputs, embedding)


# speedup vs baseline: 1.0417x; 1.0135x over previous
import jax
import jax.numpy as jnp
from jax import lax
from jax.experimental import pallas as pl
from jax.experimental.pallas import tpu as pltpu

B, S, D = 4, 4096, 1024
BLKR = 2048


def _body(in_ref, emb_ref, out_ref):
    off = lax.rem(pl.program_id(0), S // BLKR) * BLKR
    out_ref[...] = in_ref[...] + emb_ref[pl.ds(off, BLKR)]


_add = pl.pallas_call(
    _body,
    out_shape=jax.ShapeDtypeStruct((B * S, D), jnp.float32),
    grid=(B * S // BLKR,),
    in_specs=[
        pl.BlockSpec((BLKR, D), lambda i: (i, 0)),
        pl.BlockSpec((S, D), lambda i: (0, 0)),
    ],
    out_specs=pl.BlockSpec((BLKR, D), lambda i: (i, 0)),
    compiler_params=pltpu.CompilerParams(
        dimension_semantics=("arbitrary",),
    ),
)


def kernel(inputs, embedding):
    return _add(inputs.reshape(B * S, D), embedding).reshape(B, S, D)
